# baseline (speedup 1.0000x reference)
import functools

import jax
import jax.numpy as jnp
from jax import lax
from jax.experimental import pallas as pl
from jax.experimental.pallas import tpu as pltpu
from jax.experimental.pallas import tpu_sc as plsc

N = 10000
E = 320000
CHUNK = 128
NC, NS = 2, 16
NW = NC * NS
E_PAD = 327680
NCHUNKS = E_PAD // CHUNK
CH_W = NCHUNKS // NW
CH_S = NCHUNKS // NS
A_ROWS = 10240
RPS = A_ROWS // NS
PAD_SRC = 0
PAD_DST = N
MB = 1000

FH = 80
F3 = 64
F4 = 48


G = 1


def _pipelined_edge_loop(table, acc, src_v, dst_v, bufa, bufb, gsa, gsb, nchunks):
    ng = nchunks // G

    def fire(buf, sem, g):
        for k in range(G):
            pltpu.async_copy(
                table.at[src_v.at[g * G + k]],
                buf.at[pl.ds(k * CHUNK, CHUNK)], sem)

    def drain(buf, sem):
        for k in range(G):
            pltpu.make_async_copy(
                table.at[pl.ds(0, CHUNK)],
                buf.at[pl.ds(k * CHUNK, CHUNK)], sem).wait()

    def scat(buf, g):
        for k in range(G):
            pltpu.sync_copy(
                buf.at[pl.ds(k * CHUNK, CHUNK)],
                acc.at[dst_v.at[g * G + k]], add=True)

    fire(bufa, gsa, 0)

    def body(i, carry):
        ga = 2 * i
        fire(bufb, gsb, ga + 1)
        drain(bufa, gsa)
        scat(bufa, ga)

        @pl.when(ga + 2 < ng)
        def _():
            fire(bufa, gsa, ga + 2)

        drain(bufb, gsb)
        scat(bufb, ga + 1)
        return carry

    lax.fori_loop(0, ng // 2, body, 0)


@functools.cache
def _make_sc_agg(F):
    mesh = plsc.VectorSubcoreMesh(core_axis_name="c", subcore_axis_name="s")

    @functools.partial(
        pl.kernel,
        mesh=mesh,
        compiler_params=pltpu.CompilerParams(use_tc_tiling_on_sc=False),
        out_type=jax.ShapeDtypeStruct((NC, A_ROWS, F), jnp.float32),
        scratch_types=[
            pltpu.VMEM((CH_W, CHUNK), jnp.int32),
            pltpu.VMEM((CH_W, CHUNK), jnp.int32),
            pltpu.VMEM((G * CHUNK, F), jnp.float32),
            pltpu.VMEM((G * CHUNK, F), jnp.float32),
            pltpu.VMEM_SHARED((A_ROWS, F), jnp.float32),
            pltpu.SemaphoreType.DMA,
            pltpu.SemaphoreType.DMA,
        ],
    )
    def agg(table, srcc, dstc, zeros, out, src_v, dst_v, bufa, bufb, acc, gsa, gsb):
        c = lax.axis_index("c")
        s = lax.axis_index("s")
        wid = s * NC + c
        pltpu.sync_copy(zeros.at[pl.ds(s * RPS, RPS)], acc.at[pl.ds(s * RPS, RPS)])
        base = wid * CH_W
        pltpu.sync_copy(srcc.at[pl.ds(base, CH_W)], src_v)
        pltpu.sync_copy(dstc.at[pl.ds(base, CH_W)], dst_v)
        plsc.subcore_barrier()
        _pipelined_edge_loop(table, acc, src_v, dst_v, bufa, bufb, gsa, gsb, CH_W)
        plsc.subcore_barrier()
        pltpu.sync_copy(acc.at[pl.ds(s * RPS, RPS)], out.at[c, pl.ds(s * RPS, RPS)])

    return agg


@functools.cache
def _make_sc_agg_cols():
    mesh = plsc.VectorSubcoreMesh(core_axis_name="c", subcore_axis_name="s")

    @functools.partial(
        pl.kernel,
        mesh=mesh,
        compiler_params=pltpu.CompilerParams(use_tc_tiling_on_sc=False),
        out_type=jax.ShapeDtypeStruct((NC, A_ROWS, FH), jnp.float32),
        scratch_types=[
            pltpu.VMEM((CH_S, CHUNK), jnp.int32),
            pltpu.VMEM((CH_S, CHUNK), jnp.int32),
            pltpu.VMEM((G * CHUNK, FH), jnp.float32),
            pltpu.VMEM((G * CHUNK, FH), jnp.float32),
            pltpu.VMEM_SHARED((A_ROWS, FH), jnp.float32),
            pltpu.SemaphoreType.DMA,
            pltpu.SemaphoreType.DMA,
        ],
    )
    def agg(table, srcc, dstc, zeros, out, src_v, dst_v, bufa, bufb, acc, gsa, gsb):
        c = lax.axis_index("c")
        s = lax.axis_index("s")
        pltpu.sync_copy(zeros.at[pl.ds(s * RPS, RPS)], acc.at[pl.ds(s * RPS, RPS)])
        base = s * CH_S
        pltpu.sync_copy(srcc.at[pl.ds(base, CH_S)], src_v)
        pltpu.sync_copy(dstc.at[pl.ds(base, CH_S)], dst_v)
        plsc.subcore_barrier()
        _pipelined_edge_loop(table.at[c], acc, src_v, dst_v, bufa, bufb,
                             gsa, gsb, CH_S)
        plsc.subcore_barrier()
        pltpu.sync_copy(acc.at[pl.ds(s * RPS, RPS)], out.at[c, pl.ds(s * RPS, RPS)])

    return agg


def _row_spec(f):
    return pl.BlockSpec((MB, f), lambda i: (i, 0))


def _part_spec(f):
    return pl.BlockSpec((NC, MB, f), lambda i: (0, i, 0))


def _full_spec(shape):
    nd = len(shape)
    return pl.BlockSpec(shape, lambda i, _n=nd: (0,) * _n)


def _tc1_body(x_r, p_r, w1s_r, w1n_r, b1_r, w2n_r, h1_r, p2_r, inv_r):
    deg8 = p_r[1][:, 48:56]
    inv = 1.0 / jnp.maximum(deg8[:, :1], 1.0)
    hn = jnp.concatenate([p_r[0], p_r[1][:, :48]], axis=1) * inv
    h1 = jnp.maximum(
        jnp.dot(x_r[...], w1s_r[...], preferred_element_type=jnp.float32)
        + jnp.dot(hn, w1n_r[...], preferred_element_type=jnp.float32)
        + b1_r[...],
        0.0,
    )
    h1_r[...] = h1
    p2 = jnp.dot(h1, w2n_r[...], preferred_element_type=jnp.float32)
    p2_r[0, :, :] = p2[:, :FH]
    p2_r[1, :, :] = p2[:, FH:]
    inv_r[...] = jnp.broadcast_to(inv, (MB, 8))


def _tc2_body(h_r, p_r, inv_r, ws_r, b_r, wn_r, h2_r, pn_r):
    ps = jnp.concatenate([p_r[0], p_r[1]], axis=1)
    agg = ps[:, :156] * inv_r[:, :1]
    h2 = jnp.maximum(
        jnp.dot(h_r[...], ws_r[...], preferred_element_type=jnp.float32)
        + agg
        + b_r[...],
        0.0,
    )
    h2_r[...] = h2
    pn_r[...] = jnp.dot(h2, wn_r[...], preferred_element_type=jnp.float32)


def _tc3_body(h_r, p_r, inv_r, ws_r, b_r, wn_r, h2_r, pn_r):
    ps = p_r[0] + p_r[1]
    agg = ps[:, :56] * inv_r[:, :1]
    h2 = jnp.maximum(
        jnp.dot(h_r[...], ws_r[...], preferred_element_type=jnp.float32)
        + agg
        + b_r[...],
        0.0,
    )
    h2_r[...] = h2
    pn_r[...] = jnp.dot(h2, wn_r[...], preferred_element_type=jnp.float32)


def _tc4_body(h3_r, p_r, inv_r, w4s_r, b4_r, out_r, sh_acc, sa_acc):
    i = pl.program_id(0)

    @pl.when(i == 0)
    def _():
        sh_acc[...] = jnp.zeros_like(sh_acc)
        sa_acc[...] = jnp.zeros_like(sa_acc)

    ps = p_r[0] + p_r[1]
    agg = ps[:, :40] * inv_r[:, :1]
    sh_acc[...] += jnp.sum(h3_r[...], axis=0, keepdims=True)
    sa_acc[...] += jnp.sum(agg, axis=0, keepdims=True)

    @pl.when(i == pl.num_programs(0) - 1)
    def _():
        out_r[...] = (
            jnp.dot(sh_acc[...] * (1.0 / N), w4s_r[...],
                    preferred_element_type=jnp.float32)
            + sa_acc[...] * (1.0 / N)
            + b4_r[...]
        )


def _segment_partials(table, src2d, dst2d, F):
    zeros = jnp.zeros((A_ROWS, F), jnp.float32)
    return _make_sc_agg(F)(table, src2d, dst2d, zeros)


def _segment_cols(table3, src2d, dst2d):
    zeros = jnp.zeros((A_ROWS, FH), jnp.float32)
    return _make_sc_agg_cols()(table3, src2d, dst2d, zeros)


def kernel(x, W1s, W1n, b1, W2s, W2n, b2, W3s, W3n, b3, W4s, W4n, b4, edge_index):
    src = edge_index[0]
    dst = edge_index[1]
    pad = E_PAD - E
    src2d = jnp.concatenate(
        [src, jnp.full((pad,), PAD_SRC, jnp.int32)]).reshape(NCHUNKS, CHUNK)
    pad_dst = PAD_DST + (jnp.arange(pad, dtype=jnp.int32) % (A_ROWS - N))
    dst2d = jnp.concatenate([dst, pad_dst]).reshape(NCHUNKS, CHUNK)

    x1 = jnp.concatenate(
        [x[:, 80:], jnp.ones((N, 16), jnp.float32),
         jnp.zeros((N, 16), jnp.float32)], axis=1)
    table1 = jnp.stack([x[:, :80], x1])
    parts1 = _segment_cols(table1, src2d, dst2d)

    W2n_p = jnp.pad(W2n, ((0, 0), (0, 2 * FH - 156)))
    h1, p2, invd = pl.pallas_call(
        _tc1_body,
        grid=(N // MB,),
        in_specs=[
            _row_spec(128), _part_spec(FH),
            _full_spec((128, 256)), _full_spec((128, 256)), _full_spec((1, 256)),
            _full_spec((256, 2 * FH)),
        ],
        out_specs=[_row_spec(256), _part_spec(FH), _row_spec(8)],
        out_shape=[
            jax.ShapeDtypeStruct((N, 256), jnp.float32),
            jax.ShapeDtypeStruct((NC, N, FH), jnp.float32),
            jax.ShapeDtypeStruct((N, 8), jnp.float32),
        ],
    )(x, parts1, W1s, W1n, b1.reshape(1, 256), W2n_p)

    parts2 = _segment_cols(p2, src2d, dst2d)

    W3n_p = jnp.pad(W3n, ((0, 0), (0, F3 - 56)))
    h2, p3 = pl.pallas_call(
        _tc2_body,
        grid=(N // MB,),
        in_specs=[
            _row_spec(256), _part_spec(FH), _row_spec(8),
            _full_spec((256, 156)), _full_spec((1, 156)), _full_spec((156, F3)),
        ],
        out_specs=[_row_spec(156), _row_spec(F3)],
        out_shape=[
            jax.ShapeDtypeStruct((N, 156), jnp.float32),
            jax.ShapeDtypeStruct((N, F3), jnp.float32),
        ],
    )(h1, parts2, invd, W2s, b2.reshape(1, 156), W3n_p)

    parts3 = _segment_partials(p3, src2d, dst2d, F3)

    W4n_p = jnp.pad(W4n, ((0, 0), (0, F4 - 40)))
    h3, q4 = pl.pallas_call(
        _tc3_body,
        grid=(N // MB,),
        in_specs=[
            _row_spec(156), _part_spec(F3), _row_spec(8),
            _full_spec((156, 56)), _full_spec((1, 56)), _full_spec((56, F4)),
        ],
        out_specs=[_row_spec(56), _row_spec(F4)],
        out_shape=[
            jax.ShapeDtypeStruct((N, 56), jnp.float32),
            jax.ShapeDtypeStruct((N, F4), jnp.float32),
        ],
    )(h2, parts3, invd, W3s, b3.reshape(1, 56), W4n_p)

    parts4 = _segment_partials(q4, src2d, dst2d, F4)

    out = pl.pallas_call(
        _tc4_body,
        grid=(N // MB,),
        in_specs=[
            _row_spec(56), _part_spec(F4), _row_spec(8),
            _full_spec((56, 40)), _full_spec((1, 40)),
        ],
        out_specs=pl.BlockSpec((1, 40), lambda i: (0, 0)),
        out_shape=jax.ShapeDtypeStruct((1, 40), jnp.float32),
        scratch_shapes=[
            pltpu.VMEM((1, 56), jnp.float32),
            pltpu.VMEM((1, 40), jnp.float32),
        ],
    )(h3, parts4, invd, W4s, b4.reshape(1, 40))

    return out


# speedup vs baseline: 1.0455x; 1.0455x over previous
import functools

import jax
import jax.numpy as jnp
from jax import lax
from jax.experimental import pallas as pl
from jax.experimental.pallas import tpu as pltpu
from jax.experimental.pallas import tpu_sc as plsc

N = 10000
E = 320000
CHUNK = 128
NC, NS = 2, 16
NW = NC * NS
E_PAD = 327680
NCHUNKS = E_PAD // CHUNK
CH_W = NCHUNKS // NW
CH_S = NCHUNKS // NS
A_ROWS = 10240
RPS = A_ROWS // NS
PAD_SRC = 0
PAD_DST = N
MB = 1000

FH = 80
F3 = 64
F4 = 48


SHIFT = 14
MASK = (1 << SHIFT) - 1


def _ring_edge_loop(table, acc, packed_v, bufs, sidx, didx, sems, ng):
    D = len(bufs)

    def unpack(r, slot):
        for q in range(CHUNK // 16):
            p = packed_v[r, pl.ds(q * 16, 16)]
            sidx[slot][pl.ds(q * 16, 16)] = lax.shift_right_logical(p, SHIFT)
            didx[slot][pl.ds(q * 16, 16)] = lax.bitwise_and(p, MASK)

    def fire(r, slot):
        unpack(r, slot)
        pltpu.async_copy(table.at[sidx[slot]], bufs[slot], sems[slot])

    def drain(slot):
        pltpu.make_async_copy(
            table.at[pl.ds(0, CHUNK)], bufs[slot], sems[slot]).wait()

    for r in range(D - 1):
        fire(r, r)

    def body(i, carry):
        base = i * D
        for k in range(D):
            r = base + k
            drain(k)
            pltpu.sync_copy(bufs[k], acc.at[didx[k]], add=True)
            rn = r + D - 1
            slot_n = (k - 1) % D

            @pl.when(rn < ng)
            def _(rn=rn, slot_n=slot_n):
                fire(rn, slot_n)
        return carry

    lax.fori_loop(0, ng // D, body, 0)


DP = 8
DC = 5


def _sc_scratch(F, ch, depth):
    return ([pltpu.VMEM((ch, CHUNK), jnp.int32)]
            + [pltpu.VMEM((CHUNK, F), jnp.float32) for _ in range(depth)]
            + [pltpu.VMEM((CHUNK,), jnp.int32) for _ in range(2 * depth)]
            + [pltpu.VMEM_SHARED((A_ROWS, F), jnp.float32)]
            + [pltpu.SemaphoreType.DMA for _ in range(depth)])


def _split_scratch(scr, depth):
    packed_v = scr[0]
    bufs = scr[1:1 + depth]
    sidx = scr[1 + depth:1 + 2 * depth]
    didx = scr[1 + 2 * depth:1 + 3 * depth]
    acc = scr[1 + 3 * depth]
    sems = scr[2 + 3 * depth:]
    return packed_v, bufs, sidx, didx, acc, sems


@functools.cache
def _make_sc_agg(F):
    mesh = plsc.VectorSubcoreMesh(core_axis_name="c", subcore_axis_name="s")

    @functools.partial(
        pl.kernel,
        mesh=mesh,
        compiler_params=pltpu.CompilerParams(use_tc_tiling_on_sc=False),
        out_type=jax.ShapeDtypeStruct((NC, A_ROWS, F), jnp.float32),
        scratch_types=_sc_scratch(F, CH_W, DP),
    )
    def agg(table, pidx, zeros, out, *scr):
        packed_v, bufs, sidx, didx, acc, sems = _split_scratch(scr, DP)
        c = lax.axis_index("c")
        s = lax.axis_index("s")
        wid = s * NC + c
        pltpu.sync_copy(zeros.at[pl.ds(s * RPS, RPS)], acc.at[pl.ds(s * RPS, RPS)])
        pltpu.sync_copy(pidx.at[pl.ds(wid * CH_W, CH_W)], packed_v)
        plsc.subcore_barrier()
        _ring_edge_loop(table, acc, packed_v, bufs, sidx, didx, sems, CH_W)
        plsc.subcore_barrier()
        pltpu.sync_copy(acc.at[pl.ds(s * RPS, RPS)], out.at[c, pl.ds(s * RPS, RPS)])

    return agg


@functools.cache
def _make_sc_agg_cols():
    mesh = plsc.VectorSubcoreMesh(core_axis_name="c", subcore_axis_name="s")

    @functools.partial(
        pl.kernel,
        mesh=mesh,
        compiler_params=pltpu.CompilerParams(use_tc_tiling_on_sc=False),
        out_type=jax.ShapeDtypeStruct((NC, A_ROWS, FH), jnp.float32),
        scratch_types=_sc_scratch(FH, CH_S, DC),
    )
    def agg(table, pidx, zeros, out, *scr):
        packed_v, bufs, sidx, didx, acc, sems = _split_scratch(scr, DC)
        c = lax.axis_index("c")
        s = lax.axis_index("s")
        pltpu.sync_copy(zeros.at[pl.ds(s * RPS, RPS)], acc.at[pl.ds(s * RPS, RPS)])
        pltpu.sync_copy(pidx.at[pl.ds(s * CH_S, CH_S)], packed_v)
        plsc.subcore_barrier()
        _ring_edge_loop(table.at[c], acc, packed_v, bufs, sidx, didx, sems, CH_S)
        plsc.subcore_barrier()
        pltpu.sync_copy(acc.at[pl.ds(s * RPS, RPS)], out.at[c, pl.ds(s * RPS, RPS)])

    return agg


def _row_spec(f):
    return pl.BlockSpec((MB, f), lambda i: (i, 0))


def _part_spec(f):
    return pl.BlockSpec((NC, MB, f), lambda i: (0, i, 0))


def _full_spec(shape):
    nd = len(shape)
    return pl.BlockSpec(shape, lambda i, _n=nd: (0,) * _n)


def _tc1_body(x_r, p_r, w1s_r, w1n_r, b1_r, w2n_r, h1_r, p2_r, inv_r):
    deg8 = p_r[1][:, 48:56]
    inv = 1.0 / jnp.maximum(deg8[:, :1], 1.0)
    hn = jnp.concatenate([p_r[0], p_r[1][:, :48]], axis=1) * inv
    h1 = jnp.maximum(
        jnp.dot(x_r[...], w1s_r[...], preferred_element_type=jnp.float32)
        + jnp.dot(hn, w1n_r[...], preferred_element_type=jnp.float32)
        + b1_r[...],
        0.0,
    )
    h1_r[...] = h1
    p2 = jnp.dot(h1, w2n_r[...], preferred_element_type=jnp.float32)
    p2_r[0, :, :] = p2[:, :FH]
    p2_r[1, :, :] = p2[:, FH:]
    inv_r[...] = jnp.broadcast_to(inv, (MB, 8))


def _tc2_body(h_r, p_r, inv_r, ws_r, b_r, wn_r, h2_r, pn_r):
    ps = jnp.concatenate([p_r[0], p_r[1]], axis=1)
    agg = ps[:, :156] * inv_r[:, :1]
    h2 = jnp.maximum(
        jnp.dot(h_r[...], ws_r[...], preferred_element_type=jnp.float32)
        + agg
        + b_r[...],
        0.0,
    )
    h2_r[...] = h2
    pn_r[...] = jnp.dot(h2, wn_r[...], preferred_element_type=jnp.float32)


def _tc3_body(h_r, p_r, inv_r, ws_r, b_r, wn_r, h2_r, pn_r):
    ps = p_r[0] + p_r[1]
    agg = ps[:, :56] * inv_r[:, :1]
    h2 = jnp.maximum(
        jnp.dot(h_r[...], ws_r[...], preferred_element_type=jnp.float32)
        + agg
        + b_r[...],
        0.0,
    )
    h2_r[...] = h2
    pn_r[...] = jnp.dot(h2, wn_r[...], preferred_element_type=jnp.float32)


def _tc4_body(h3_r, p_r, inv_r, w4s_r, b4_r, out_r, sh_acc, sa_acc):
    i = pl.program_id(0)

    @pl.when(i == 0)
    def _():
        sh_acc[...] = jnp.zeros_like(sh_acc)
        sa_acc[...] = jnp.zeros_like(sa_acc)

    ps = p_r[0] + p_r[1]
    agg = ps[:, :40] * inv_r[:, :1]
    sh_acc[...] += jnp.sum(h3_r[...], axis=0, keepdims=True)
    sa_acc[...] += jnp.sum(agg, axis=0, keepdims=True)

    @pl.when(i == pl.num_programs(0) - 1)
    def _():
        out_r[...] = (
            jnp.dot(sh_acc[...] * (1.0 / N), w4s_r[...],
                    preferred_element_type=jnp.float32)
            + sa_acc[...] * (1.0 / N)
            + b4_r[...]
        )


def _segment_partials(table, pidx2d, F):
    zeros = jnp.zeros((A_ROWS, F), jnp.float32)
    return _make_sc_agg(F)(table, pidx2d, zeros)


def _segment_cols(table3, pidx2d):
    zeros = jnp.zeros((A_ROWS, FH), jnp.float32)
    return _make_sc_agg_cols()(table3, pidx2d, zeros)


def kernel(x, W1s, W1n, b1, W2s, W2n, b2, W3s, W3n, b3, W4s, W4n, b4, edge_index):
    src = edge_index[0]
    dst = edge_index[1]
    pad = E_PAD - E
    srcp = jnp.concatenate([src, jnp.full((pad,), PAD_SRC, jnp.int32)])
    pad_dst = PAD_DST + (jnp.arange(pad, dtype=jnp.int32) % (A_ROWS - N))
    dstp = jnp.concatenate([dst, pad_dst])
    pidx2d = ((srcp << SHIFT) | dstp).reshape(NCHUNKS, CHUNK)

    x1 = jnp.concatenate(
        [x[:, 80:], jnp.ones((N, 16), jnp.float32),
         jnp.zeros((N, 16), jnp.float32)], axis=1)
    table1 = jnp.stack([x[:, :80], x1])
    parts1 = _segment_cols(table1, pidx2d)

    W2n_p = jnp.pad(W2n, ((0, 0), (0, 2 * FH - 156)))
    h1, p2, invd = pl.pallas_call(
        _tc1_body,
        grid=(N // MB,),
        in_specs=[
            _row_spec(128), _part_spec(FH),
            _full_spec((128, 256)), _full_spec((128, 256)), _full_spec((1, 256)),
            _full_spec((256, 2 * FH)),
        ],
        out_specs=[_row_spec(256), _part_spec(FH), _row_spec(8)],
        out_shape=[
            jax.ShapeDtypeStruct((N, 256), jnp.float32),
            jax.ShapeDtypeStruct((NC, N, FH), jnp.float32),
            jax.ShapeDtypeStruct((N, 8), jnp.float32),
        ],
    )(x, parts1, W1s, W1n, b1.reshape(1, 256), W2n_p)

    parts2 = _segment_cols(p2, pidx2d)

    W3n_p = jnp.pad(W3n, ((0, 0), (0, F3 - 56)))
    h2, p3 = pl.pallas_call(
        _tc2_body,
        grid=(N // MB,),
        in_specs=[
            _row_spec(256), _part_spec(FH), _row_spec(8),
            _full_spec((256, 156)), _full_spec((1, 156)), _full_spec((156, F3)),
        ],
        out_specs=[_row_spec(156), _row_spec(F3)],
        out_shape=[
            jax.ShapeDtypeStruct((N, 156), jnp.float32),
            jax.ShapeDtypeStruct((N, F3), jnp.float32),
        ],
    )(h1, parts2, invd, W2s, b2.reshape(1, 156), W3n_p)

    parts3 = _segment_partials(p3, pidx2d, F3)

    W4n_p = jnp.pad(W4n, ((0, 0), (0, F4 - 40)))
    h3, q4 = pl.pallas_call(
        _tc3_body,
        grid=(N // MB,),
        in_specs=[
            _row_spec(156), _part_spec(F3), _row_spec(8),
            _full_spec((156, 56)), _full_spec((1, 56)), _full_spec((56, F4)),
        ],
        out_specs=[_row_spec(56), _row_spec(F4)],
        out_shape=[
            jax.ShapeDtypeStruct((N, 56), jnp.float32),
            jax.ShapeDtypeStruct((N, F4), jnp.float32),
        ],
    )(h2, parts3, invd, W3s, b3.reshape(1, 56), W4n_p)

    parts4 = _segment_partials(q4, pidx2d, F4)

    out = pl.pallas_call(
        _tc4_body,
        grid=(N // MB,),
        in_specs=[
            _row_spec(56), _part_spec(F4), _row_spec(8),
            _full_spec((56, 40)), _full_spec((1, 40)),
        ],
        out_specs=pl.BlockSpec((1, 40), lambda i: (0, 0)),
        out_shape=jax.ShapeDtypeStruct((1, 40), jnp.float32),
        scratch_shapes=[
            pltpu.VMEM((1, 56), jnp.float32),
            pltpu.VMEM((1, 40), jnp.float32),
        ],
    )(h3, parts4, invd, W4s, b4.reshape(1, 40))

    return out


# speedup vs baseline: 1.4171x; 1.3554x over previous
import functools

import jax
import jax.numpy as jnp
from jax import lax
from jax.experimental import pallas as pl
from jax.experimental.pallas import tpu as pltpu
from jax.experimental.pallas import tpu_sc as plsc

N = 10000
E = 320000
CHUNK = 128
NC, NS = 2, 16
NW = NC * NS
E_PAD = 327680
NCHUNKS = E_PAD // CHUNK
CH_W = NCHUNKS // NW
CH_S = NCHUNKS // NS
A_ROWS = 10240
RPS = A_ROWS // NS
PAD_SRC = 0
PAD_DST = N
MB = 1000

FH = 80
F3 = 64
F4 = 48


SHIFT = 14
MASK = (1 << SHIFT) - 1


def _ring_edge_loop(table, acc, packed_v, bufs, sidx, didx, sems, ng):
    D = len(bufs)

    def unpack(r, slot):
        for q in range(CHUNK // 16):
            p = packed_v[r, pl.ds(q * 16, 16)]
            sidx[slot][pl.ds(q * 16, 16)] = lax.shift_right_logical(p, SHIFT)
            didx[slot][pl.ds(q * 16, 16)] = lax.bitwise_and(p, MASK)

    def fire(r, slot):
        unpack(r, slot)
        pltpu.async_copy(table.at[sidx[slot]], bufs[slot], sems[slot])

    def drain(slot):
        pltpu.make_async_copy(
            table.at[pl.ds(0, CHUNK)], bufs[slot], sems[slot]).wait()

    for r in range(D - 1):
        fire(r, r)

    def body(i, carry):
        base = i * D
        for k in range(D):
            r = base + k
            drain(k)
            pltpu.sync_copy(bufs[k], acc.at[didx[k]], add=True)
            rn = r + D - 1
            slot_n = (k - 1) % D

            @pl.when(rn < ng)
            def _(rn=rn, slot_n=slot_n):
                fire(rn, slot_n)
        return carry

    lax.fori_loop(0, ng // D, body, 0)


DP = 8
DC = 8


def _sc_scratch(F, ch, depth, dtype):
    return ([pltpu.VMEM((ch, CHUNK), jnp.int32)]
            + [pltpu.VMEM((CHUNK, F), dtype) for _ in range(depth)]
            + [pltpu.VMEM((CHUNK,), jnp.int32) for _ in range(2 * depth)]
            + [pltpu.VMEM_SHARED((A_ROWS, F), dtype)]
            + [pltpu.SemaphoreType.DMA for _ in range(depth)])


def _split_scratch(scr, depth):
    packed_v = scr[0]
    bufs = scr[1:1 + depth]
    sidx = scr[1 + depth:1 + 2 * depth]
    didx = scr[1 + 2 * depth:1 + 3 * depth]
    acc = scr[1 + 3 * depth]
    sems = scr[2 + 3 * depth:]
    return packed_v, bufs, sidx, didx, acc, sems


@functools.cache
def _make_sc_agg(F):
    mesh = plsc.VectorSubcoreMesh(core_axis_name="c", subcore_axis_name="s")

    @functools.partial(
        pl.kernel,
        mesh=mesh,
        compiler_params=pltpu.CompilerParams(use_tc_tiling_on_sc=False),
        out_type=jax.ShapeDtypeStruct((NC, A_ROWS, F), jnp.float32),
        scratch_types=_sc_scratch(F, CH_W, DP, jnp.float32),
    )
    def agg(table, pidx, zeros, out, *scr):
        packed_v, bufs, sidx, didx, acc, sems = _split_scratch(scr, DP)
        c = lax.axis_index("c")
        s = lax.axis_index("s")
        wid = s * NC + c
        pltpu.sync_copy(zeros.at[pl.ds(s * RPS, RPS)], acc.at[pl.ds(s * RPS, RPS)])
        pltpu.sync_copy(pidx.at[pl.ds(wid * CH_W, CH_W)], packed_v)
        plsc.subcore_barrier()
        _ring_edge_loop(table, acc, packed_v, bufs, sidx, didx, sems, CH_W)
        plsc.subcore_barrier()
        pltpu.sync_copy(acc.at[pl.ds(s * RPS, RPS)], out.at[c, pl.ds(s * RPS, RPS)])

    return agg


@functools.cache
def _make_sc_agg_cols():
    mesh = plsc.VectorSubcoreMesh(core_axis_name="c", subcore_axis_name="s")

    @functools.partial(
        pl.kernel,
        mesh=mesh,
        compiler_params=pltpu.CompilerParams(use_tc_tiling_on_sc=False),
        out_type=jax.ShapeDtypeStruct((NC, A_ROWS, FH), jnp.bfloat16),
        scratch_types=_sc_scratch(FH, CH_S, DC, jnp.bfloat16),
    )
    def agg(table, pidx, zeros, out, *scr):
        packed_v, bufs, sidx, didx, acc, sems = _split_scratch(scr, DC)
        c = lax.axis_index("c")
        s = lax.axis_index("s")
        pltpu.sync_copy(zeros.at[pl.ds(s * RPS, RPS)], acc.at[pl.ds(s * RPS, RPS)])
        pltpu.sync_copy(pidx.at[pl.ds(s * CH_S, CH_S)], packed_v)
        plsc.subcore_barrier()
        _ring_edge_loop(table.at[c], acc, packed_v, bufs, sidx, didx, sems, CH_S)
        plsc.subcore_barrier()
        pltpu.sync_copy(acc.at[pl.ds(s * RPS, RPS)], out.at[c, pl.ds(s * RPS, RPS)])

    return agg


def _row_spec(f):
    return pl.BlockSpec((MB, f), lambda i: (i, 0))


def _part_spec(f):
    return pl.BlockSpec((NC, MB, f), lambda i: (0, i, 0))


def _full_spec(shape):
    nd = len(shape)
    return pl.BlockSpec(shape, lambda i, _n=nd: (0,) * _n)


def _tc1_body(x_r, p_r, w1s_r, w1n_r, b1_r, w2n_r, h1_r, p2_r, inv_r):
    pf = p_r[...].astype(jnp.float32)
    deg8 = pf[1][:, 48:56]
    inv = 1.0 / jnp.maximum(deg8[:, :1], 1.0)
    hn = jnp.concatenate([pf[0], pf[1][:, :48]], axis=1) * inv
    h1 = jnp.maximum(
        jnp.dot(x_r[...], w1s_r[...], preferred_element_type=jnp.float32)
        + jnp.dot(hn, w1n_r[...], preferred_element_type=jnp.float32)
        + b1_r[...],
        0.0,
    )
    h1_r[...] = h1
    p2 = jnp.dot(h1, w2n_r[...],
                 preferred_element_type=jnp.float32).astype(jnp.bfloat16)
    p2_r[0, :, :] = p2[:, :FH]
    p2_r[1, :, :] = p2[:, FH:]
    inv_r[...] = jnp.broadcast_to(inv, (MB, 8))


def _tc2_body(h_r, p_r, inv_r, ws_r, b_r, wn_r, h2_r, pn_r):
    pf = p_r[...].astype(jnp.float32)
    ps = jnp.concatenate([pf[0], pf[1]], axis=1)
    agg = ps[:, :156] * inv_r[:, :1]
    h2 = jnp.maximum(
        jnp.dot(h_r[...], ws_r[...], preferred_element_type=jnp.float32)
        + agg
        + b_r[...],
        0.0,
    )
    h2_r[...] = h2
    pn_r[...] = jnp.dot(h2, wn_r[...], preferred_element_type=jnp.float32)


def _tc3_body(h_r, p_r, inv_r, ws_r, b_r, wn_r, h2_r, pn_r):
    ps = p_r[0] + p_r[1]
    agg = ps[:, :56] * inv_r[:, :1]
    h2 = jnp.maximum(
        jnp.dot(h_r[...], ws_r[...], preferred_element_type=jnp.float32)
        + agg
        + b_r[...],
        0.0,
    )
    h2_r[...] = h2
    pn_r[...] = jnp.dot(h2, wn_r[...], preferred_element_type=jnp.float32)


def _tc4_body(h3_r, p_r, inv_r, w4s_r, b4_r, out_r, sh_acc, sa_acc):
    i = pl.program_id(0)

    @pl.when(i == 0)
    def _():
        sh_acc[...] = jnp.zeros_like(sh_acc)
        sa_acc[...] = jnp.zeros_like(sa_acc)

    ps = p_r[0] + p_r[1]
    agg = ps[:, :40] * inv_r[:, :1]
    sh_acc[...] += jnp.sum(h3_r[...], axis=0, keepdims=True)
    sa_acc[...] += jnp.sum(agg, axis=0, keepdims=True)

    @pl.when(i == pl.num_programs(0) - 1)
    def _():
        out_r[...] = (
            jnp.dot(sh_acc[...] * (1.0 / N), w4s_r[...],
                    preferred_element_type=jnp.float32)
            + sa_acc[...] * (1.0 / N)
            + b4_r[...]
        )


def _segment_partials(table, pidx2d, F):
    zeros = jnp.zeros((A_ROWS, F), jnp.float32)
    return _make_sc_agg(F)(table, pidx2d, zeros)


def _segment_cols(table3, pidx2d):
    zeros = jnp.zeros((A_ROWS, FH), jnp.bfloat16)
    return _make_sc_agg_cols()(table3, pidx2d, zeros)


def kernel(x, W1s, W1n, b1, W2s, W2n, b2, W3s, W3n, b3, W4s, W4n, b4, edge_index):
    src = edge_index[0]
    dst = edge_index[1]
    pad = E_PAD - E
    srcp = jnp.concatenate([src, jnp.full((pad,), PAD_SRC, jnp.int32)])
    pad_dst = PAD_DST + (jnp.arange(pad, dtype=jnp.int32) % (A_ROWS - N))
    dstp = jnp.concatenate([dst, pad_dst])
    pidx2d = ((srcp << SHIFT) | dstp).reshape(NCHUNKS, CHUNK)

    x1 = jnp.concatenate(
        [x[:, 80:], jnp.ones((N, 16), jnp.float32),
         jnp.zeros((N, 16), jnp.float32)], axis=1)
    table1 = jnp.stack([x[:, :80], x1]).astype(jnp.bfloat16)
    parts1 = _segment_cols(table1, pidx2d)

    W2n_p = jnp.pad(W2n, ((0, 0), (0, 2 * FH - 156)))
    h1, p2, invd = pl.pallas_call(
        _tc1_body,
        grid=(N // MB,),
        in_specs=[
            _row_spec(128), _part_spec(FH),
            _full_spec((128, 256)), _full_spec((128, 256)), _full_spec((1, 256)),
            _full_spec((256, 2 * FH)),
        ],
        out_specs=[_row_spec(256), _part_spec(FH), _row_spec(8)],
        out_shape=[
            jax.ShapeDtypeStruct((N, 256), jnp.float32),
            jax.ShapeDtypeStruct((NC, N, FH), jnp.bfloat16),
            jax.ShapeDtypeStruct((N, 8), jnp.float32),
        ],
    )(x, parts1, W1s, W1n, b1.reshape(1, 256), W2n_p)

    parts2 = _segment_cols(p2, pidx2d)

    W3n_p = jnp.pad(W3n, ((0, 0), (0, F3 - 56)))
    h2, p3 = pl.pallas_call(
        _tc2_body,
        grid=(N // MB,),
        in_specs=[
            _row_spec(256), _part_spec(FH), _row_spec(8),
            _full_spec((256, 156)), _full_spec((1, 156)), _full_spec((156, F3)),
        ],
        out_specs=[_row_spec(156), _row_spec(F3)],
        out_shape=[
            jax.ShapeDtypeStruct((N, 156), jnp.float32),
            jax.ShapeDtypeStruct((N, F3), jnp.float32),
        ],
    )(h1, parts2, invd, W2s, b2.reshape(1, 156), W3n_p)

    parts3 = _segment_partials(p3, pidx2d, F3)

    W4n_p = jnp.pad(W4n, ((0, 0), (0, F4 - 40)))
    h3, q4 = pl.pallas_call(
        _tc3_body,
        grid=(N // MB,),
        in_specs=[
            _row_spec(156), _part_spec(F3), _row_spec(8),
            _full_spec((156, 56)), _full_spec((1, 56)), _full_spec((56, F4)),
        ],
        out_specs=[_row_spec(56), _row_spec(F4)],
        out_shape=[
            jax.ShapeDtypeStruct((N, 56), jnp.float32),
            jax.ShapeDtypeStruct((N, F4), jnp.float32),
        ],
    )(h2, parts3, invd, W3s, b3.reshape(1, 56), W4n_p)

    parts4 = _segment_partials(q4, pidx2d, F4)

    out = pl.pallas_call(
        _tc4_body,
        grid=(N // MB,),
        in_specs=[
            _row_spec(56), _part_spec(F4), _row_spec(8),
            _full_spec((56, 40)), _full_spec((1, 40)),
        ],
        out_specs=pl.BlockSpec((1, 40), lambda i: (0, 0)),
        out_shape=jax.ShapeDtypeStruct((1, 40), jnp.float32),
        scratch_shapes=[
            pltpu.VMEM((1, 56), jnp.float32),
            pltpu.VMEM((1, 40), jnp.float32),
        ],
    )(h3, parts4, invd, W4s, b4.reshape(1, 40))

    return out


# speedup vs baseline: 1.7251x; 1.2174x over previous
import functools

import jax
import jax.numpy as jnp
from jax import lax
from jax.experimental import pallas as pl
from jax.experimental.pallas import tpu as pltpu
from jax.experimental.pallas import tpu_sc as plsc

N = 10000
E = 320000
CHUNK = 128
NC, NS = 2, 16
NW = NC * NS
E_PAD = 327680
NCHUNKS = E_PAD // CHUNK
CH_W = NCHUNKS // NW
CH_S = NCHUNKS // NS
A_ROWS = 10240
RPS = A_ROWS // NS
PAD_SRC = 0
PAD_DST = N
MB = 1000

FH = 80
F3 = 64
F4 = 48


SHIFT = 14
MASK = (1 << SHIFT) - 1


def _ring_edge_loop(table, acc, packed_v, bufs, sidx, didx, sems, ng):
    D = len(bufs)

    def unpack(r, slot):
        for q in range(CHUNK // 16):
            p = packed_v[r, pl.ds(q * 16, 16)]
            sidx[slot][pl.ds(q * 16, 16)] = lax.shift_right_logical(p, SHIFT)
            didx[slot][pl.ds(q * 16, 16)] = lax.bitwise_and(p, MASK)

    def fire(r, slot):
        unpack(r, slot)
        pltpu.async_copy(table.at[sidx[slot]], bufs[slot], sems[slot])

    def drain(slot):
        pltpu.make_async_copy(
            table.at[pl.ds(0, CHUNK)], bufs[slot], sems[slot]).wait()

    for r in range(D - 1):
        fire(r, r)

    def body(i, carry):
        base = i * D
        for k in range(D):
            r = base + k
            drain(k)
            pltpu.sync_copy(bufs[k], acc.at[didx[k]], add=True)
            rn = r + D - 1
            slot_n = (k - 1) % D

            @pl.when(rn < ng)
            def _(rn=rn, slot_n=slot_n):
                fire(rn, slot_n)
        return carry

    lax.fori_loop(0, ng // D, body, 0)


DP = 8
DC = 8


def _sc_scratch(F, ch, depth, dtype):
    return ([pltpu.VMEM((ch, CHUNK), jnp.int32)]
            + [pltpu.VMEM((CHUNK, F), dtype) for _ in range(depth)]
            + [pltpu.VMEM((CHUNK,), jnp.int32) for _ in range(2 * depth)]
            + [pltpu.VMEM_SHARED((A_ROWS, F), dtype)]
            + [pltpu.SemaphoreType.DMA for _ in range(depth)])


def _split_scratch(scr, depth):
    packed_v = scr[0]
    bufs = scr[1:1 + depth]
    sidx = scr[1 + depth:1 + 2 * depth]
    didx = scr[1 + 2 * depth:1 + 3 * depth]
    acc = scr[1 + 3 * depth]
    sems = scr[2 + 3 * depth:]
    return packed_v, bufs, sidx, didx, acc, sems


@functools.cache
def _make_sc_agg(F):
    mesh = plsc.VectorSubcoreMesh(core_axis_name="c", subcore_axis_name="s")

    @functools.partial(
        pl.kernel,
        mesh=mesh,
        compiler_params=pltpu.CompilerParams(use_tc_tiling_on_sc=False),
        out_type=jax.ShapeDtypeStruct((NC, A_ROWS, F), jnp.bfloat16),
        scratch_types=_sc_scratch(F, CH_W, DP, jnp.bfloat16),
    )
    def agg(table, pidx, zeros, out, *scr):
        packed_v, bufs, sidx, didx, acc, sems = _split_scratch(scr, DP)
        c = lax.axis_index("c")
        s = lax.axis_index("s")
        wid = s * NC + c
        pltpu.sync_copy(zeros.at[pl.ds(s * RPS, RPS)], acc.at[pl.ds(s * RPS, RPS)])
        pltpu.sync_copy(pidx.at[pl.ds(wid * CH_W, CH_W)], packed_v)
        plsc.subcore_barrier()
        _ring_edge_loop(table, acc, packed_v, bufs, sidx, didx, sems, CH_W)
        plsc.subcore_barrier()
        pltpu.sync_copy(acc.at[pl.ds(s * RPS, RPS)], out.at[c, pl.ds(s * RPS, RPS)])

    return agg


@functools.cache
def _make_sc_agg_cols():
    mesh = plsc.VectorSubcoreMesh(core_axis_name="c", subcore_axis_name="s")

    @functools.partial(
        pl.kernel,
        mesh=mesh,
        compiler_params=pltpu.CompilerParams(use_tc_tiling_on_sc=False),
        out_type=jax.ShapeDtypeStruct((NC, A_ROWS, FH), jnp.bfloat16),
        scratch_types=_sc_scratch(FH, CH_S, DC, jnp.bfloat16),
    )
    def agg(table, pidx, zeros, out, *scr):
        packed_v, bufs, sidx, didx, acc, sems = _split_scratch(scr, DC)
        c = lax.axis_index("c")
        s = lax.axis_index("s")
        pltpu.sync_copy(zeros.at[pl.ds(s * RPS, RPS)], acc.at[pl.ds(s * RPS, RPS)])
        pltpu.sync_copy(pidx.at[pl.ds(s * CH_S, CH_S)], packed_v)
        plsc.subcore_barrier()
        _ring_edge_loop(table.at[c], acc, packed_v, bufs, sidx, didx, sems, CH_S)
        plsc.subcore_barrier()
        pltpu.sync_copy(acc.at[pl.ds(s * RPS, RPS)], out.at[c, pl.ds(s * RPS, RPS)])

    return agg


def _row_spec(f):
    return pl.BlockSpec((MB, f), lambda i: (i, 0))


def _part_spec(f):
    return pl.BlockSpec((NC, MB, f), lambda i: (0, i, 0))


def _full_spec(shape):
    nd = len(shape)
    return pl.BlockSpec(shape, lambda i, _n=nd: (0,) * _n)


def _tc1_body(x_r, p_r, w1s_r, w1n_r, b1_r, w2n_r, h1_r, p2_r, inv_r):
    pf = p_r[...].astype(jnp.float32)
    deg8 = pf[1][:, 48:56]
    inv = 1.0 / jnp.maximum(deg8[:, :1], 1.0)
    hn = jnp.concatenate([pf[0], pf[1][:, :48]], axis=1) * inv
    h1 = jnp.maximum(
        jnp.dot(x_r[...], w1s_r[...], preferred_element_type=jnp.float32)
        + jnp.dot(hn, w1n_r[...], preferred_element_type=jnp.float32)
        + b1_r[...],
        0.0,
    )
    h1_r[...] = h1
    p2 = jnp.dot(h1, w2n_r[...],
                 preferred_element_type=jnp.float32).astype(jnp.bfloat16)
    p2_r[0, :, :] = p2[:, :FH]
    p2_r[1, :, :] = p2[:, FH:]
    inv_r[...] = jnp.broadcast_to(inv, (MB, 8))


def _tc2_body(h_r, p_r, inv_r, ws_r, b_r, wn_r, h2_r, pn_r):
    pf = p_r[...].astype(jnp.float32)
    ps = jnp.concatenate([pf[0], pf[1]], axis=1)
    agg = ps[:, :156] * inv_r[:, :1]
    h2 = jnp.maximum(
        jnp.dot(h_r[...], ws_r[...], preferred_element_type=jnp.float32)
        + agg
        + b_r[...],
        0.0,
    )
    h2_r[...] = h2
    pn_r[...] = jnp.dot(h2, wn_r[...],
                        preferred_element_type=jnp.float32).astype(jnp.bfloat16)


def _tc3_body(h_r, p_r, inv_r, ws_r, b_r, wn_r, h2_r, pn_r):
    pf = p_r[...].astype(jnp.float32)
    ps = pf[0] + pf[1]
    agg = ps[:, :56] * inv_r[:, :1]
    h2 = jnp.maximum(
        jnp.dot(h_r[...], ws_r[...], preferred_element_type=jnp.float32)
        + agg
        + b_r[...],
        0.0,
    )
    h2_r[...] = h2
    pn_r[...] = jnp.dot(h2, wn_r[...],
                        preferred_element_type=jnp.float32).astype(jnp.bfloat16)


def _tc4_body(h3_r, p_r, inv_r, w4s_r, b4_r, out_r, sh_acc, sa_acc):
    i = pl.program_id(0)

    @pl.when(i == 0)
    def _():
        sh_acc[...] = jnp.zeros_like(sh_acc)
        sa_acc[...] = jnp.zeros_like(sa_acc)

    pf = p_r[...].astype(jnp.float32)
    ps = pf[0] + pf[1]
    agg = ps[:, :40] * inv_r[:, :1]
    sh_acc[...] += jnp.sum(h3_r[...], axis=0, keepdims=True)
    sa_acc[...] += jnp.sum(agg, axis=0, keepdims=True)

    @pl.when(i == pl.num_programs(0) - 1)
    def _():
        out_r[...] = (
            jnp.dot(sh_acc[...] * (1.0 / N), w4s_r[...],
                    preferred_element_type=jnp.float32)
            + sa_acc[...] * (1.0 / N)
            + b4_r[...]
        )


def _segment_partials(table, pidx2d, F):
    zeros = jnp.zeros((A_ROWS, F), jnp.bfloat16)
    return _make_sc_agg(F)(table, pidx2d, zeros)


def _segment_cols(table3, pidx2d):
    zeros = jnp.zeros((A_ROWS, FH), jnp.bfloat16)
    return _make_sc_agg_cols()(table3, pidx2d, zeros)


def kernel(x, W1s, W1n, b1, W2s, W2n, b2, W3s, W3n, b3, W4s, W4n, b4, edge_index):
    src = edge_index[0]
    dst = edge_index[1]
    pad = E_PAD - E
    srcp = jnp.concatenate([src, jnp.full((pad,), PAD_SRC, jnp.int32)])
    pad_dst = PAD_DST + (jnp.arange(pad, dtype=jnp.int32) % (A_ROWS - N))
    dstp = jnp.concatenate([dst, pad_dst])
    pidx2d = ((srcp << SHIFT) | dstp).reshape(NCHUNKS, CHUNK)

    x1 = jnp.concatenate(
        [x[:, 80:], jnp.ones((N, 16), jnp.float32),
         jnp.zeros((N, 16), jnp.float32)], axis=1)
    table1 = jnp.stack([x[:, :80], x1]).astype(jnp.bfloat16)
    parts1 = _segment_cols(table1, pidx2d)

    W2n_p = jnp.pad(W2n, ((0, 0), (0, 2 * FH - 156)))
    h1, p2, invd = pl.pallas_call(
        _tc1_body,
        grid=(N // MB,),
        in_specs=[
            _row_spec(128), _part_spec(FH),
            _full_spec((128, 256)), _full_spec((128, 256)), _full_spec((1, 256)),
            _full_spec((256, 2 * FH)),
        ],
        out_specs=[_row_spec(256), _part_spec(FH), _row_spec(8)],
        out_shape=[
            jax.ShapeDtypeStruct((N, 256), jnp.float32),
            jax.ShapeDtypeStruct((NC, N, FH), jnp.bfloat16),
            jax.ShapeDtypeStruct((N, 8), jnp.float32),
        ],
    )(x, parts1, W1s, W1n, b1.reshape(1, 256), W2n_p)

    parts2 = _segment_cols(p2, pidx2d)

    W3n_p = jnp.pad(W3n, ((0, 0), (0, F3 - 56)))
    h2, p3 = pl.pallas_call(
        _tc2_body,
        grid=(N // MB,),
        in_specs=[
            _row_spec(256), _part_spec(FH), _row_spec(8),
            _full_spec((256, 156)), _full_spec((1, 156)), _full_spec((156, F3)),
        ],
        out_specs=[_row_spec(156), _row_spec(F3)],
        out_shape=[
            jax.ShapeDtypeStruct((N, 156), jnp.float32),
            jax.ShapeDtypeStruct((N, F3), jnp.bfloat16),
        ],
    )(h1, parts2, invd, W2s, b2.reshape(1, 156), W3n_p)

    parts3 = _segment_partials(p3, pidx2d, F3)

    W4n_p = jnp.pad(W4n, ((0, 0), (0, F4 - 40)))
    h3, q4 = pl.pallas_call(
        _tc3_body,
        grid=(N // MB,),
        in_specs=[
            _row_spec(156), _part_spec(F3), _row_spec(8),
            _full_spec((156, 56)), _full_spec((1, 56)), _full_spec((56, F4)),
        ],
        out_specs=[_row_spec(56), _row_spec(F4)],
        out_shape=[
            jax.ShapeDtypeStruct((N, 56), jnp.float32),
            jax.ShapeDtypeStruct((N, F4), jnp.bfloat16),
        ],
    )(h2, parts3, invd, W3s, b3.reshape(1, 56), W4n_p)

    parts4 = _segment_partials(q4, pidx2d, F4)

    out = pl.pallas_call(
        _tc4_body,
        grid=(N // MB,),
        in_specs=[
            _row_spec(56), _part_spec(F4), _row_spec(8),
            _full_spec((56, 40)), _full_spec((1, 40)),
        ],
        out_specs=pl.BlockSpec((1, 40), lambda i: (0, 0)),
        out_shape=jax.ShapeDtypeStruct((1, 40), jnp.float32),
        scratch_shapes=[
            pltpu.VMEM((1, 56), jnp.float32),
            pltpu.VMEM((1, 40), jnp.float32),
        ],
    )(h3, parts4, invd, W4s, b4.reshape(1, 40))

    return out


# speedup vs baseline: 3.0296x; 1.7562x over previous
import functools

import jax
import jax.numpy as jnp
from jax import lax
from jax.experimental import pallas as pl
from jax.experimental.pallas import tpu as pltpu
from jax.experimental.pallas import tpu_sc as plsc

N = 10000
E = 320000
CHUNK = 128
NC, NS = 2, 16
NW = NC * NS
E_PAD = 327680
NCHUNKS = E_PAD // CHUNK
CH_W = NCHUNKS // NW
CH_S = NCHUNKS // NS
A_ROWS = 10240
RPS = A_ROWS // NS
PAD_SRC = 0
PAD_DST = N
MB = 1000

FH = 80
F3 = 64
F4 = 48


SHIFT = 14
MASK = (1 << SHIFT) - 1


def _ring_edge_loop(table, acc, packed_v, bufs, sidx, didx, sems, ng):
    D = len(bufs)

    def unpack(r, slot):
        for q in range(CHUNK // 16):
            p = packed_v[r, pl.ds(q * 16, 16)]
            sidx[slot][pl.ds(q * 16, 16)] = lax.shift_right_logical(p, SHIFT)
            didx[slot][pl.ds(q * 16, 16)] = lax.bitwise_and(p, MASK)

    def fire(r, slot):
        unpack(r, slot)
        pltpu.async_copy(table.at[sidx[slot]], bufs[slot], sems[slot])

    def drain(slot):
        pltpu.make_async_copy(
            table.at[pl.ds(0, CHUNK)], bufs[slot], sems[slot]).wait()

    for r in range(D - 1):
        fire(r, r)

    def body(i, carry):
        base = i * D
        for k in range(D):
            r = base + k
            drain(k)
            pltpu.sync_copy(bufs[k], acc.at[didx[k]], add=True)
            rn = r + D - 1
            slot_n = (k - 1) % D

            @pl.when(rn < ng)
            def _(rn=rn, slot_n=slot_n):
                fire(rn, slot_n)
        return carry

    lax.fori_loop(0, ng // D, body, 0)


DP = 8
DC = 8


RPT = N // NS


def _sc_scratch(F, ch, depth, dtype):
    return ([pltpu.VMEM((ch, CHUNK), jnp.int32)]
            + [pltpu.VMEM((CHUNK, F), dtype) for _ in range(depth)]
            + [pltpu.VMEM((CHUNK,), jnp.int32) for _ in range(2 * depth)]
            + [pltpu.VMEM_SHARED((A_ROWS, F), dtype)]
            + [pltpu.VMEM_SHARED((N, F), dtype)]
            + [pltpu.SemaphoreType.DMA for _ in range(depth)])


def _split_scratch(scr, depth):
    packed_v = scr[0]
    bufs = scr[1:1 + depth]
    sidx = scr[1 + depth:1 + 2 * depth]
    didx = scr[1 + 2 * depth:1 + 3 * depth]
    acc = scr[1 + 3 * depth]
    tbl = scr[2 + 3 * depth]
    sems = scr[3 + 3 * depth:]
    return packed_v, bufs, sidx, didx, acc, tbl, sems


@functools.cache
def _make_sc_agg(F):
    mesh = plsc.VectorSubcoreMesh(core_axis_name="c", subcore_axis_name="s")

    @functools.partial(
        pl.kernel,
        mesh=mesh,
        compiler_params=pltpu.CompilerParams(use_tc_tiling_on_sc=False),
        out_type=jax.ShapeDtypeStruct((NC, A_ROWS, F), jnp.bfloat16),
        scratch_types=_sc_scratch(F, CH_W, DP, jnp.bfloat16),
    )
    def agg(table, pidx, zeros, out, *scr):
        packed_v, bufs, sidx, didx, acc, tbl, sems = _split_scratch(scr, DP)
        c = lax.axis_index("c")
        s = lax.axis_index("s")
        wid = s * NC + c
        pltpu.sync_copy(zeros.at[pl.ds(s * RPS, RPS)], acc.at[pl.ds(s * RPS, RPS)])
        pltpu.sync_copy(table.at[pl.ds(s * RPT, RPT)], tbl.at[pl.ds(s * RPT, RPT)])
        pltpu.sync_copy(pidx.at[pl.ds(wid * CH_W, CH_W)], packed_v)
        plsc.subcore_barrier()
        _ring_edge_loop(tbl, acc, packed_v, bufs, sidx, didx, sems, CH_W)
        plsc.subcore_barrier()
        pltpu.sync_copy(acc.at[pl.ds(s * RPS, RPS)], out.at[c, pl.ds(s * RPS, RPS)])

    return agg


@functools.cache
def _make_sc_agg_cols():
    mesh = plsc.VectorSubcoreMesh(core_axis_name="c", subcore_axis_name="s")

    @functools.partial(
        pl.kernel,
        mesh=mesh,
        compiler_params=pltpu.CompilerParams(use_tc_tiling_on_sc=False),
        out_type=jax.ShapeDtypeStruct((NC, A_ROWS, FH), jnp.bfloat16),
        scratch_types=_sc_scratch(FH, CH_S, DC, jnp.bfloat16),
    )
    def agg(table, pidx, zeros, out, *scr):
        packed_v, bufs, sidx, didx, acc, tbl, sems = _split_scratch(scr, DC)
        c = lax.axis_index("c")
        s = lax.axis_index("s")
        pltpu.sync_copy(zeros.at[pl.ds(s * RPS, RPS)], acc.at[pl.ds(s * RPS, RPS)])
        pltpu.sync_copy(table.at[c, pl.ds(s * RPT, RPT)],
                        tbl.at[pl.ds(s * RPT, RPT)])
        pltpu.sync_copy(pidx.at[pl.ds(s * CH_S, CH_S)], packed_v)
        plsc.subcore_barrier()
        _ring_edge_loop(tbl, acc, packed_v, bufs, sidx, didx, sems, CH_S)
        plsc.subcore_barrier()
        pltpu.sync_copy(acc.at[pl.ds(s * RPS, RPS)], out.at[c, pl.ds(s * RPS, RPS)])

    return agg


def _row_spec(f):
    return pl.BlockSpec((MB, f), lambda i: (i, 0))


def _part_spec(f):
    return pl.BlockSpec((NC, MB, f), lambda i: (0, i, 0))


def _full_spec(shape):
    nd = len(shape)
    return pl.BlockSpec(shape, lambda i, _n=nd: (0,) * _n)


def _tc1_body(x_r, p_r, w1s_r, w1n_r, b1_r, w2n_r, h1_r, p2_r, inv_r):
    pf = p_r[...].astype(jnp.float32)
    deg8 = pf[1][:, 48:56]
    inv = 1.0 / jnp.maximum(deg8[:, :1], 1.0)
    hn = jnp.concatenate([pf[0], pf[1][:, :48]], axis=1) * inv
    h1 = jnp.maximum(
        jnp.dot(x_r[...], w1s_r[...], preferred_element_type=jnp.float32)
        + jnp.dot(hn, w1n_r[...], preferred_element_type=jnp.float32)
        + b1_r[...],
        0.0,
    )
    h1_r[...] = h1
    p2 = jnp.dot(h1, w2n_r[...],
                 preferred_element_type=jnp.float32).astype(jnp.bfloat16)
    p2_r[0, :, :] = p2[:, :FH]
    p2_r[1, :, :] = p2[:, FH:]
    inv_r[...] = jnp.broadcast_to(inv, (MB, 8))


def _tc2_body(h_r, p_r, inv_r, ws_r, b_r, wn_r, h2_r, pn_r):
    pf = p_r[...].astype(jnp.float32)
    ps = jnp.concatenate([pf[0], pf[1]], axis=1)
    agg = ps[:, :156] * inv_r[:, :1]
    h2 = jnp.maximum(
        jnp.dot(h_r[...], ws_r[...], preferred_element_type=jnp.float32)
        + agg
        + b_r[...],
        0.0,
    )
    h2_r[...] = h2
    pn_r[...] = jnp.dot(h2, wn_r[...],
                        preferred_element_type=jnp.float32).astype(jnp.bfloat16)


def _tc3_body(h_r, p_r, inv_r, ws_r, b_r, wn_r, h2_r, pn_r):
    pf = p_r[...].astype(jnp.float32)
    ps = pf[0] + pf[1]
    agg = ps[:, :56] * inv_r[:, :1]
    h2 = jnp.maximum(
        jnp.dot(h_r[...], ws_r[...], preferred_element_type=jnp.float32)
        + agg
        + b_r[...],
        0.0,
    )
    h2_r[...] = h2
    pn_r[...] = jnp.dot(h2, wn_r[...],
                        preferred_element_type=jnp.float32).astype(jnp.bfloat16)


def _tc4_body(h3_r, p_r, inv_r, w4s_r, b4_r, out_r, sh_acc, sa_acc):
    i = pl.program_id(0)

    @pl.when(i == 0)
    def _():
        sh_acc[...] = jnp.zeros_like(sh_acc)
        sa_acc[...] = jnp.zeros_like(sa_acc)

    pf = p_r[...].astype(jnp.float32)
    ps = pf[0] + pf[1]
    agg = ps[:, :40] * inv_r[:, :1]
    sh_acc[...] += jnp.sum(h3_r[...], axis=0, keepdims=True)
    sa_acc[...] += jnp.sum(agg, axis=0, keepdims=True)

    @pl.when(i == pl.num_programs(0) - 1)
    def _():
        out_r[...] = (
            jnp.dot(sh_acc[...] * (1.0 / N), w4s_r[...],
                    preferred_element_type=jnp.float32)
            + sa_acc[...] * (1.0 / N)
            + b4_r[...]
        )


def _segment_partials(table, pidx2d, F):
    zeros = jnp.zeros((A_ROWS, F), jnp.bfloat16)
    return _make_sc_agg(F)(table, pidx2d, zeros)


def _segment_cols(table3, pidx2d):
    zeros = jnp.zeros((A_ROWS, FH), jnp.bfloat16)
    return _make_sc_agg_cols()(table3, pidx2d, zeros)


def kernel(x, W1s, W1n, b1, W2s, W2n, b2, W3s, W3n, b3, W4s, W4n, b4, edge_index):
    src = edge_index[0]
    dst = edge_index[1]
    pad = E_PAD - E
    srcp = jnp.concatenate([src, jnp.full((pad,), PAD_SRC, jnp.int32)])
    pad_dst = PAD_DST + (jnp.arange(pad, dtype=jnp.int32) % (A_ROWS - N))
    dstp = jnp.concatenate([dst, pad_dst])
    pidx2d = ((srcp << SHIFT) | dstp).reshape(NCHUNKS, CHUNK)

    x1 = jnp.concatenate(
        [x[:, 80:], jnp.ones((N, 16), jnp.float32),
         jnp.zeros((N, 16), jnp.float32)], axis=1)
    table1 = jnp.stack([x[:, :80], x1]).astype(jnp.bfloat16)
    parts1 = _segment_cols(table1, pidx2d)

    W2n_p = jnp.pad(W2n, ((0, 0), (0, 2 * FH - 156)))
    h1, p2, invd = pl.pallas_call(
        _tc1_body,
        grid=(N // MB,),
        in_specs=[
            _row_spec(128), _part_spec(FH),
            _full_spec((128, 256)), _full_spec((128, 256)), _full_spec((1, 256)),
            _full_spec((256, 2 * FH)),
        ],
        out_specs=[_row_spec(256), _part_spec(FH), _row_spec(8)],
        out_shape=[
            jax.ShapeDtypeStruct((N, 256), jnp.float32),
            jax.ShapeDtypeStruct((NC, N, FH), jnp.bfloat16),
            jax.ShapeDtypeStruct((N, 8), jnp.float32),
        ],
    )(x, parts1, W1s, W1n, b1.reshape(1, 256), W2n_p)

    parts2 = _segment_cols(p2, pidx2d)

    W3n_p = jnp.pad(W3n, ((0, 0), (0, F3 - 56)))
    h2, p3 = pl.pallas_call(
        _tc2_body,
        grid=(N // MB,),
        in_specs=[
            _row_spec(256), _part_spec(FH), _row_spec(8),
            _full_spec((256, 156)), _full_spec((1, 156)), _full_spec((156, F3)),
        ],
        out_specs=[_row_spec(156), _row_spec(F3)],
        out_shape=[
            jax.ShapeDtypeStruct((N, 156), jnp.float32),
            jax.ShapeDtypeStruct((N, F3), jnp.bfloat16),
        ],
    )(h1, parts2, invd, W2s, b2.reshape(1, 156), W3n_p)

    parts3 = _segment_partials(p3, pidx2d, F3)

    W4n_p = jnp.pad(W4n, ((0, 0), (0, F4 - 40)))
    h3, q4 = pl.pallas_call(
        _tc3_body,
        grid=(N // MB,),
        in_specs=[
            _row_spec(156), _part_spec(F3), _row_spec(8),
            _full_spec((156, 56)), _full_spec((1, 56)), _full_spec((56, F4)),
        ],
        out_specs=[_row_spec(56), _row_spec(F4)],
        out_shape=[
            jax.ShapeDtypeStruct((N, 56), jnp.float32),
            jax.ShapeDtypeStruct((N, F4), jnp.bfloat16),
        ],
    )(h2, parts3, invd, W3s, b3.reshape(1, 56), W4n_p)

    parts4 = _segment_partials(q4, pidx2d, F4)

    out = pl.pallas_call(
        _tc4_body,
        grid=(N // MB,),
        in_specs=[
            _row_spec(56), _part_spec(F4), _row_spec(8),
            _full_spec((56, 40)), _full_spec((1, 40)),
        ],
        out_specs=pl.BlockSpec((1, 40), lambda i: (0, 0)),
        out_shape=jax.ShapeDtypeStruct((1, 40), jnp.float32),
        scratch_shapes=[
            pltpu.VMEM((1, 56), jnp.float32),
            pltpu.VMEM((1, 40), jnp.float32),
        ],
    )(h3, parts4, invd, W4s, b4.reshape(1, 40))

    return out


# speedup vs baseline: 3.2271x; 1.0652x over previous
import functools

import jax
import jax.numpy as jnp
from jax import lax
from jax.experimental import pallas as pl
from jax.experimental.pallas import tpu as pltpu
from jax.experimental.pallas import tpu_sc as plsc

N = 10000
E = 320000
CHUNK = 128
NC, NS = 2, 16
NW = NC * NS
E_PAD = 327680
NCHUNKS = E_PAD // CHUNK
CH_W = NCHUNKS // NW
CH_S = NCHUNKS // NS
A_ROWS = 10240
RPS = A_ROWS // NS
PAD_SRC = 0
PAD_DST = N
MB = 1000

FH = 80
F3 = 64
F4 = 48


SHIFT = 14
MASK = (1 << SHIFT) - 1


def _ring_edge_loop(table, acc, packed_v, bufs, sidx, didx, gsems, ssems, ng):
    D = len(bufs)

    def unpack(r, slot):
        for q in range(CHUNK // 16):
            p = packed_v[r, pl.ds(q * 16, 16)]
            sidx[slot][pl.ds(q * 16, 16)] = lax.shift_right_logical(p, SHIFT)
            didx[slot][pl.ds(q * 16, 16)] = lax.bitwise_and(p, MASK)

    def fire(r, slot):
        unpack(r, slot)
        pltpu.async_copy(table.at[sidx[slot]], bufs[slot], gsems[slot])

    def drain_gather(slot):
        pltpu.make_async_copy(
            table.at[pl.ds(0, CHUNK)], bufs[slot], gsems[slot]).wait()

    def drain_scatter(slot):
        pltpu.make_async_copy(
            bufs[slot], acc.at[pl.ds(0, CHUNK)], ssems[slot]).wait()

    for r in range(D - 1):
        fire(r, r)

    def body(i, carry):
        base = i * D
        for k in range(D):
            r = base + k
            drain_gather(k)
            pltpu.async_copy(bufs[k], acc.at[didx[k]], ssems[k], add=True)
            rn = r + D - 1
            slot_n = (k - 1) % D

            if k == 0:
                @pl.when((i > 0) & (rn < ng))
                def _(rn=rn, slot_n=slot_n):
                    drain_scatter(slot_n)
                    fire(rn, slot_n)

                @pl.when((i == 0) & (rn < ng))
                def _(rn=rn, slot_n=slot_n):
                    fire(rn, slot_n)
            else:
                @pl.when(rn < ng)
                def _(rn=rn, slot_n=slot_n):
                    drain_scatter(slot_n)
                    fire(rn, slot_n)
        return carry

    lax.fori_loop(0, ng // D, body, 0)
    for slot in range(D):
        drain_scatter(slot)


DP = 8
DC = 8


RPT = N // NS


def _sc_scratch(F, ch, depth, dtype):
    return ([pltpu.VMEM((ch, CHUNK), jnp.int32)]
            + [pltpu.VMEM((CHUNK, F), dtype) for _ in range(depth)]
            + [pltpu.VMEM((CHUNK,), jnp.int32) for _ in range(2 * depth)]
            + [pltpu.VMEM_SHARED((A_ROWS, F), dtype)]
            + [pltpu.VMEM_SHARED((N, F), dtype)]
            + [pltpu.SemaphoreType.DMA for _ in range(2 * depth)])


def _split_scratch(scr, depth):
    packed_v = scr[0]
    bufs = scr[1:1 + depth]
    sidx = scr[1 + depth:1 + 2 * depth]
    didx = scr[1 + 2 * depth:1 + 3 * depth]
    acc = scr[1 + 3 * depth]
    tbl = scr[2 + 3 * depth]
    gsems = scr[3 + 3 * depth:3 + 3 * depth + depth]
    ssems = scr[3 + 4 * depth:]
    return packed_v, bufs, sidx, didx, acc, tbl, gsems, ssems


@functools.cache
def _make_sc_agg(F):
    mesh = plsc.VectorSubcoreMesh(core_axis_name="c", subcore_axis_name="s")

    @functools.partial(
        pl.kernel,
        mesh=mesh,
        compiler_params=pltpu.CompilerParams(use_tc_tiling_on_sc=False),
        out_type=jax.ShapeDtypeStruct((NC, A_ROWS, F), jnp.bfloat16),
        scratch_types=_sc_scratch(F, CH_W, DP, jnp.bfloat16),
    )
    def agg(table, pidx, zeros, out, *scr):
        packed_v, bufs, sidx, didx, acc, tbl, gsems, ssems = _split_scratch(scr, DP)
        c = lax.axis_index("c")
        s = lax.axis_index("s")
        wid = s * NC + c
        pltpu.sync_copy(zeros.at[pl.ds(s * RPS, RPS)], acc.at[pl.ds(s * RPS, RPS)])
        pltpu.sync_copy(table.at[pl.ds(s * RPT, RPT)], tbl.at[pl.ds(s * RPT, RPT)])
        pltpu.sync_copy(pidx.at[pl.ds(wid * CH_W, CH_W)], packed_v)
        plsc.subcore_barrier()
        _ring_edge_loop(tbl, acc, packed_v, bufs, sidx, didx, gsems, ssems, CH_W)
        plsc.subcore_barrier()
        pltpu.sync_copy(acc.at[pl.ds(s * RPS, RPS)], out.at[c, pl.ds(s * RPS, RPS)])

    return agg


@functools.cache
def _make_sc_agg_cols():
    mesh = plsc.VectorSubcoreMesh(core_axis_name="c", subcore_axis_name="s")

    @functools.partial(
        pl.kernel,
        mesh=mesh,
        compiler_params=pltpu.CompilerParams(use_tc_tiling_on_sc=False),
        out_type=jax.ShapeDtypeStruct((NC, A_ROWS, FH), jnp.bfloat16),
        scratch_types=_sc_scratch(FH, CH_S, DC, jnp.bfloat16),
    )
    def agg(table, pidx, zeros, out, *scr):
        packed_v, bufs, sidx, didx, acc, tbl, gsems, ssems = _split_scratch(scr, DC)
        c = lax.axis_index("c")
        s = lax.axis_index("s")
        pltpu.sync_copy(zeros.at[pl.ds(s * RPS, RPS)], acc.at[pl.ds(s * RPS, RPS)])
        pltpu.sync_copy(table.at[c, pl.ds(s * RPT, RPT)],
                        tbl.at[pl.ds(s * RPT, RPT)])
        pltpu.sync_copy(pidx.at[pl.ds(s * CH_S, CH_S)], packed_v)
        plsc.subcore_barrier()
        _ring_edge_loop(tbl, acc, packed_v, bufs, sidx, didx, gsems, ssems, CH_S)
        plsc.subcore_barrier()
        pltpu.sync_copy(acc.at[pl.ds(s * RPS, RPS)], out.at[c, pl.ds(s * RPS, RPS)])

    return agg


def _row_spec(f):
    return pl.BlockSpec((MB, f), lambda i: (i, 0))


def _part_spec(f):
    return pl.BlockSpec((NC, MB, f), lambda i: (0, i, 0))


def _full_spec(shape):
    nd = len(shape)
    return pl.BlockSpec(shape, lambda i, _n=nd: (0,) * _n)


def _tc1_body(x_r, p_r, w1s_r, w1n_r, b1_r, w2n_r, h1_r, p2_r, inv_r):
    pf = p_r[...].astype(jnp.float32)
    deg8 = pf[1][:, 48:56]
    inv = 1.0 / jnp.maximum(deg8[:, :1], 1.0)
    hn = jnp.concatenate([pf[0], pf[1][:, :48]], axis=1) * inv
    h1 = jnp.maximum(
        jnp.dot(x_r[...], w1s_r[...], preferred_element_type=jnp.float32)
        + jnp.dot(hn, w1n_r[...], preferred_element_type=jnp.float32)
        + b1_r[...],
        0.0,
    )
    h1_r[...] = h1
    p2 = jnp.dot(h1, w2n_r[...],
                 preferred_element_type=jnp.float32).astype(jnp.bfloat16)
    p2_r[0, :, :] = p2[:, :FH]
    p2_r[1, :, :] = p2[:, FH:]
    inv_r[...] = jnp.broadcast_to(inv, (MB, 8))


def _tc2_body(h_r, p_r, inv_r, ws_r, b_r, wn_r, h2_r, pn_r):
    pf = p_r[...].astype(jnp.float32)
    ps = jnp.concatenate([pf[0], pf[1]], axis=1)
    agg = ps[:, :156] * inv_r[:, :1]
    h2 = jnp.maximum(
        jnp.dot(h_r[...], ws_r[...], preferred_element_type=jnp.float32)
        + agg
        + b_r[...],
        0.0,
    )
    h2_r[...] = h2
    pn_r[...] = jnp.dot(h2, wn_r[...],
                        preferred_element_type=jnp.float32).astype(jnp.bfloat16)


def _tc3_body(h_r, p_r, inv_r, ws_r, b_r, wn_r, h2_r, pn_r):
    pf = p_r[...].astype(jnp.float32)
    ps = pf[0] + pf[1]
    agg = ps[:, :56] * inv_r[:, :1]
    h2 = jnp.maximum(
        jnp.dot(h_r[...], ws_r[...], preferred_element_type=jnp.float32)
        + agg
        + b_r[...],
        0.0,
    )
    h2_r[...] = h2
    pn_r[...] = jnp.dot(h2, wn_r[...],
                        preferred_element_type=jnp.float32).astype(jnp.bfloat16)


def _tc4_body(h3_r, p_r, inv_r, w4s_r, b4_r, out_r, sh_acc, sa_acc):
    i = pl.program_id(0)

    @pl.when(i == 0)
    def _():
        sh_acc[...] = jnp.zeros_like(sh_acc)
        sa_acc[...] = jnp.zeros_like(sa_acc)

    pf = p_r[...].astype(jnp.float32)
    ps = pf[0] + pf[1]
    agg = ps[:, :40] * inv_r[:, :1]
    sh_acc[...] += jnp.sum(h3_r[...], axis=0, keepdims=True)
    sa_acc[...] += jnp.sum(agg, axis=0, keepdims=True)

    @pl.when(i == pl.num_programs(0) - 1)
    def _():
        out_r[...] = (
            jnp.dot(sh_acc[...] * (1.0 / N), w4s_r[...],
                    preferred_element_type=jnp.float32)
            + sa_acc[...] * (1.0 / N)
            + b4_r[...]
        )


def _segment_partials(table, pidx2d, F):
    zeros = jnp.zeros((A_ROWS, F), jnp.bfloat16)
    return _make_sc_agg(F)(table, pidx2d, zeros)


def _segment_cols(table3, pidx2d):
    zeros = jnp.zeros((A_ROWS, FH), jnp.bfloat16)
    return _make_sc_agg_cols()(table3, pidx2d, zeros)


def kernel(x, W1s, W1n, b1, W2s, W2n, b2, W3s, W3n, b3, W4s, W4n, b4, edge_index):
    src = edge_index[0]
    dst = edge_index[1]
    pad = E_PAD - E
    srcp = jnp.concatenate([src, jnp.full((pad,), PAD_SRC, jnp.int32)])
    pad_dst = PAD_DST + (jnp.arange(pad, dtype=jnp.int32) % (A_ROWS - N))
    dstp = jnp.concatenate([dst, pad_dst])
    pidx2d = ((srcp << SHIFT) | dstp).reshape(NCHUNKS, CHUNK)

    x1 = jnp.concatenate(
        [x[:, 80:], jnp.ones((N, 16), jnp.float32),
         jnp.zeros((N, 16), jnp.float32)], axis=1)
    table1 = jnp.stack([x[:, :80], x1]).astype(jnp.bfloat16)
    parts1 = _segment_cols(table1, pidx2d)

    W2n_p = jnp.pad(W2n, ((0, 0), (0, 2 * FH - 156)))
    h1, p2, invd = pl.pallas_call(
        _tc1_body,
        grid=(N // MB,),
        in_specs=[
            _row_spec(128), _part_spec(FH),
            _full_spec((128, 256)), _full_spec((128, 256)), _full_spec((1, 256)),
            _full_spec((256, 2 * FH)),
        ],
        out_specs=[_row_spec(256), _part_spec(FH), _row_spec(8)],
        out_shape=[
            jax.ShapeDtypeStruct((N, 256), jnp.float32),
            jax.ShapeDtypeStruct((NC, N, FH), jnp.bfloat16),
            jax.ShapeDtypeStruct((N, 8), jnp.float32),
        ],
    )(x, parts1, W1s, W1n, b1.reshape(1, 256), W2n_p)

    parts2 = _segment_cols(p2, pidx2d)

    W3n_p = jnp.pad(W3n, ((0, 0), (0, F3 - 56)))
    h2, p3 = pl.pallas_call(
        _tc2_body,
        grid=(N // MB,),
        in_specs=[
            _row_spec(256), _part_spec(FH), _row_spec(8),
            _full_spec((256, 156)), _full_spec((1, 156)), _full_spec((156, F3)),
        ],
        out_specs=[_row_spec(156), _row_spec(F3)],
        out_shape=[
            jax.ShapeDtypeStruct((N, 156), jnp.float32),
            jax.ShapeDtypeStruct((N, F3), jnp.bfloat16),
        ],
    )(h1, parts2, invd, W2s, b2.reshape(1, 156), W3n_p)

    parts3 = _segment_partials(p3, pidx2d, F3)

    W4n_p = jnp.pad(W4n, ((0, 0), (0, F4 - 40)))
    h3, q4 = pl.pallas_call(
        _tc3_body,
        grid=(N // MB,),
        in_specs=[
            _row_spec(156), _part_spec(F3), _row_spec(8),
            _full_spec((156, 56)), _full_spec((1, 56)), _full_spec((56, F4)),
        ],
        out_specs=[_row_spec(56), _row_spec(F4)],
        out_shape=[
            jax.ShapeDtypeStruct((N, 56), jnp.float32),
            jax.ShapeDtypeStruct((N, F4), jnp.bfloat16),
        ],
    )(h2, parts3, invd, W3s, b3.reshape(1, 56), W4n_p)

    parts4 = _segment_partials(q4, pidx2d, F4)

    out = pl.pallas_call(
        _tc4_body,
        grid=(N // MB,),
        in_specs=[
            _row_spec(56), _part_spec(F4), _row_spec(8),
            _full_spec((56, 40)), _full_spec((1, 40)),
        ],
        out_specs=pl.BlockSpec((1, 40), lambda i: (0, 0)),
        out_shape=jax.ShapeDtypeStruct((1, 40), jnp.float32),
        scratch_shapes=[
            pltpu.VMEM((1, 56), jnp.float32),
            pltpu.VMEM((1, 40), jnp.float32),
        ],
    )(h3, parts4, invd, W4s, b4.reshape(1, 40))

    return out
